# baseline (speedup 1.0000x reference)
import functools

import jax
import jax.numpy as jnp
from jax import lax
from jax.experimental import pallas as pl
from jax.experimental.pallas import tpu as pltpu
from jax.experimental.pallas import tpu_sc as plsc

N = 10000
H = 64
NC = 2
NS = 16
NW = NC * NS
CHUNK = 128
CPW = 80
EPAD = NW * CPW * CHUNK
NPAD = 10112
DUMMY = N
RPW = NPAD // NS

_mesh = plsc.VectorSubcoreMesh(core_axis_name="c", subcore_axis_name="s")
_sc_params = pltpu.CompilerParams(use_tc_tiling_on_sc=False,
                                 needs_layout_passes=False)
_f32 = jnp.float32


@functools.partial(
    pl.kernel,
    out_type=jax.ShapeDtypeStruct((NW, NPAD), _f32),
    mesh=_mesh,
    compiler_params=_sc_params,
    scratch_types=[
        pltpu.VMEM((CPW, CHUNK), jnp.int32),
        pltpu.VMEM((NPAD,), _f32),
    ],
)
def _deg_sc(dst_hbm, zeros_hbm, out_hbm, dst_v, hist):
    c = lax.axis_index("c")
    s = lax.axis_index("s")
    wid = c * NS + s

    pltpu.sync_copy(zeros_hbm, hist)
    pltpu.sync_copy(dst_hbm.at[wid], dst_v)
    ones_v = jnp.ones((16,), _f32)

    def outer(j, carry):
        for k in range(CHUNK // 16):
            idx = dst_v[j, pl.ds(16 * k, 16)]
            plsc.addupdate_scatter(hist, [idx], ones_v)
        return carry

    lax.fori_loop(0, CPW, outer, 0)
    pltpu.sync_copy(hist, out_hbm.at[wid])


@functools.partial(
    pl.kernel,
    out_type=jax.ShapeDtypeStruct((NC, NPAD, H), _f32),
    mesh=_mesh,
    compiler_params=_sc_params,
    scratch_types=[
        pltpu.VMEM((CPW, CHUNK), jnp.int32),
        pltpu.VMEM((CPW, CHUNK), jnp.int32),
        pltpu.VMEM((4, CHUNK, H), _f32),
        pltpu.VMEM_SHARED((NPAD, H), _f32),
        pltpu.SemaphoreType.DMA,
        pltpu.SemaphoreType.DMA,
        pltpu.SemaphoreType.DMA,
        pltpu.SemaphoreType.DMA,
    ],
)
def _agg_sc(hs_hbm, src_hbm, dst_hbm, zeros_hbm, out_hbm,
            src_v, dst_v, rb, acc, g0, g1, g2, g3):
    c = lax.axis_index("c")
    s = lax.axis_index("s")
    wid = c * NS + s
    r0 = s * RPW
    gsem = [g0, g1, g2, g3]
    table = hs_hbm

    pltpu.sync_copy(zeros_hbm, acc.at[pl.ds(r0, RPW)])
    pltpu.sync_copy(src_hbm.at[wid], src_v)
    pltpu.sync_copy(dst_hbm.at[wid], dst_v)
    plsc.subcore_barrier()

    pltpu.async_copy(table.at[src_v.at[0]], rb.at[0], g0)
    pltpu.async_copy(table.at[src_v.at[1]], rb.at[1], g1)

    def outer(t, carry):
        for b in range(4):
            j = 4 * t + b
            pltpu.make_async_copy(
                table.at[src_v.at[j]], rb.at[b], gsem[b]).wait()
            pltpu.sync_copy(rb.at[b], acc.at[dst_v.at[j]], add=True)
            b2 = (b + 2) % 4

            @pl.when(j + 2 < CPW)
            def _():
                pltpu.async_copy(
                    table.at[src_v.at[j + 2]], rb.at[b2], gsem[b2])
        return carry

    lax.fori_loop(0, CPW // 4, outer, 0)
    plsc.subcore_barrier()
    pltpu.sync_copy(acc.at[pl.ds(r0, RPW)], out_hbm.at[c].at[pl.ds(r0, RPW)])


def _enc_tc_body(x_ref, we_ref, be_ref, cnt_ref, h_ref, hs_ref, dinv_ref):
    deg = 1.0 + jnp.sum(cnt_ref[:, :N], axis=0)[:, None]
    dinv = lax.rsqrt(deg)
    h = jnp.dot(x_ref[...], we_ref[...],
                preferred_element_type=_f32) + be_ref[...]
    h_ref[...] = h
    hs_ref[...] = h * dinv
    dinv_ref[...] = dinv


def _mid_tc_body(agg_ref, h_ref, dinv_ref, w_ref, b_ref, h_o, hs_o):
    dinv = dinv_ref[...]
    ssum = agg_ref[0, :N, :] + agg_ref[1, :N, :]
    m = ssum * dinv + h_ref[...] * (dinv * dinv)
    h = jnp.maximum(
        jnp.dot(m, w_ref[...], preferred_element_type=_f32) + b_ref[...],
        0.0)
    h_o[...] = h
    hs_o[...] = h * dinv


def _fin_tc_body(h_ref, wo1_ref, bo1_ref, wo2_ref, bo2_ref, out_ref):
    g = jnp.mean(h_ref[...], axis=0, keepdims=True)
    hid = jnp.maximum(
        jnp.dot(g, wo1_ref[...], preferred_element_type=_f32)
        + bo1_ref[...], 0.0)
    out_ref[...] = (
        jnp.dot(hid, wo2_ref[...], preferred_element_type=_f32)
        + bo2_ref[...])


_enc_tc = pl.pallas_call(
    _enc_tc_body,
    out_shape=[
        jax.ShapeDtypeStruct((N, H), _f32),
        jax.ShapeDtypeStruct((N, H), _f32),
        jax.ShapeDtypeStruct((N, 1), _f32),
    ],
)

_mid_tc = pl.pallas_call(
    _mid_tc_body,
    out_shape=[
        jax.ShapeDtypeStruct((N, H), _f32),
        jax.ShapeDtypeStruct((N, H), _f32),
    ],
)

_fin_tc = pl.pallas_call(
    _fin_tc_body,
    out_shape=jax.ShapeDtypeStruct((1, 1), _f32),
)


def kernel(x, edge_index, edge_attr, W_enc, b_enc, W_edge, b_edge,
           W_g0, b_g0, W_g1, b_g1, W_g2, b_g2, W_o1, b_o1, W_o2, b_o2):
    del edge_attr, W_edge, b_edge
    src = edge_index[0].astype(jnp.int32)
    dst = edge_index[1].astype(jnp.int32)
    pad = EPAD - src.shape[0]
    src_p = jnp.concatenate(
        [src, jnp.zeros((pad,), jnp.int32)]).reshape(NW, CPW, CHUNK)
    dst_p = jnp.concatenate(
        [dst, jnp.full((pad,), DUMMY, jnp.int32)]).reshape(NW, CPW, CHUNK)

    zeros1 = jnp.zeros((NPAD,), _f32)
    zeros64 = jnp.zeros((RPW, H), _f32)

    cnt = _deg_sc(dst_p, zeros1)
    h, hs, dinv = _enc_tc(x, W_enc, b_enc.reshape(1, H), cnt)

    for W_l, b_l in ((W_g0, b_g0), (W_g1, b_g1), (W_g2, b_g2)):
        agg = _agg_sc(hs, src_p, dst_p, zeros64)
        h, hs = _mid_tc(agg, h, dinv, W_l, b_l.reshape(1, H))

    return _fin_tc(h, W_o1, b_o1.reshape(1, H // 2), W_o2, b_o2.reshape(1, 1))


# speedup vs baseline: 2.5550x; 2.5550x over previous
import functools

import jax
import jax.numpy as jnp
from jax import lax
from jax.experimental import pallas as pl
from jax.experimental.pallas import tpu as pltpu
from jax.experimental.pallas import tpu_sc as plsc

N = 10000
H = 64
NC = 2
NS = 16
NW = NC * NS
CHUNK = 128
CPW = 80
EPAD = NW * CPW * CHUNK
NPAD = 10112
DUMMY = N
RPW = NPAD // NS
CPW_LAST = 20

_mesh = plsc.VectorSubcoreMesh(core_axis_name="c", subcore_axis_name="s")
_sc_params = pltpu.CompilerParams(use_tc_tiling_on_sc=False,
                                 needs_layout_passes=False)
_f32 = jnp.float32


@functools.partial(
    pl.kernel,
    out_type=jax.ShapeDtypeStruct((NW, NPAD), _f32),
    mesh=_mesh,
    compiler_params=_sc_params,
    scratch_types=[
        pltpu.VMEM((CPW, CHUNK), jnp.int32),
        pltpu.VMEM((NPAD,), _f32),
    ],
)
def _deg_sc(dst_hbm, zeros_hbm, out_hbm, dst_v, hist):
    c = lax.axis_index("c")
    s = lax.axis_index("s")
    wid = c * NS + s

    pltpu.sync_copy(zeros_hbm, hist)
    pltpu.sync_copy(dst_hbm.at[wid], dst_v)
    ones_v = jnp.ones((16,), _f32)
    limit = jnp.where(wid == NW - 1, CPW_LAST, CPW)

    def outer(j, carry):
        for k in range(CHUNK // 16):
            idx = dst_v[j, pl.ds(16 * k, 16)]
            plsc.addupdate_scatter(hist, [idx], ones_v)
        return carry

    lax.fori_loop(0, limit, outer, 0)
    pltpu.sync_copy(hist, out_hbm.at[wid])


@functools.partial(
    pl.kernel,
    out_type=jax.ShapeDtypeStruct((NC, NPAD, H), _f32),
    mesh=_mesh,
    compiler_params=_sc_params,
    scratch_types=[
        pltpu.VMEM((CPW, CHUNK), jnp.int32),
        pltpu.VMEM((CPW, CHUNK), jnp.int32),
        pltpu.VMEM((4, CHUNK, H), _f32),
        pltpu.VMEM_SHARED((NPAD, H), _f32),
        pltpu.SemaphoreType.DMA,
        pltpu.SemaphoreType.DMA,
        pltpu.SemaphoreType.DMA,
        pltpu.SemaphoreType.DMA,
    ],
)
def _agg_sc(hs_hbm, src_hbm, dst_hbm, zeros_hbm, out_hbm,
            src_v, dst_v, rb, acc, g0, g1, g2, g3):
    c = lax.axis_index("c")
    s = lax.axis_index("s")
    wid = c * NS + s
    r0 = s * RPW
    gsem = [g0, g1, g2, g3]
    table = hs_hbm

    pltpu.sync_copy(zeros_hbm, acc.at[pl.ds(r0, RPW)])
    pltpu.sync_copy(src_hbm.at[wid], src_v)
    pltpu.sync_copy(dst_hbm.at[wid], dst_v)
    plsc.subcore_barrier()

    pltpu.async_copy(table.at[src_v.at[0]], rb.at[0], g0)
    pltpu.async_copy(table.at[src_v.at[1]], rb.at[1], g1)

    limit = jnp.where(wid == NW - 1, CPW_LAST, CPW)

    def outer(t, carry):
        for b in range(4):
            j = 4 * t + b
            pltpu.make_async_copy(
                table.at[src_v.at[j]], rb.at[b], gsem[b]).wait()
            pltpu.sync_copy(rb.at[b], acc.at[dst_v.at[j]], add=True)
            b2 = (b + 2) % 4

            @pl.when(j + 2 < limit)
            def _():
                pltpu.async_copy(
                    table.at[src_v.at[j + 2]], rb.at[b2], gsem[b2])
        return carry

    lax.fori_loop(0, limit // 4, outer, 0)
    plsc.subcore_barrier()
    pltpu.sync_copy(acc.at[pl.ds(r0, RPW)], out_hbm.at[c].at[pl.ds(r0, RPW)])


def _enc_tc_body(x_ref, we_ref, be_ref, cnt_ref, h_ref, hs_ref, dinv_ref):
    deg = 1.0 + jnp.sum(cnt_ref[:, :N], axis=0)[:, None]
    dinv = 1.0 / jnp.sqrt(deg)
    h = jnp.dot(x_ref[...], we_ref[...],
                preferred_element_type=_f32) + be_ref[...]
    h_ref[...] = h
    hs_ref[...] = h * dinv
    dinv_ref[...] = dinv


def _mid_tc_body(agg_ref, hs_ref, dinv_ref, w_ref, b_ref, h_o, hs_o):
    dinv = dinv_ref[...]
    ssum = (agg_ref[0, :N, :] + agg_ref[1, :N, :]) + hs_ref[...]
    m = ssum * dinv
    h = jnp.maximum(
        jnp.dot(m, w_ref[...], preferred_element_type=_f32) + b_ref[...],
        0.0)
    h_o[...] = h
    hs_o[...] = h * dinv


def _fin_tc_body(h_ref, wo1_ref, bo1_ref, wo2_ref, bo2_ref, out_ref):
    g = jnp.mean(h_ref[...], axis=0, keepdims=True)
    hid = jnp.maximum(
        jnp.dot(g, wo1_ref[...], preferred_element_type=_f32)
        + bo1_ref[...], 0.0)
    out_ref[...] = (
        jnp.dot(hid, wo2_ref[...], preferred_element_type=_f32)
        + bo2_ref[...])


_enc_tc = pl.pallas_call(
    _enc_tc_body,
    out_shape=[
        jax.ShapeDtypeStruct((N, H), _f32),
        jax.ShapeDtypeStruct((N, H), _f32),
        jax.ShapeDtypeStruct((N, 1), _f32),
    ],
)

_mid_tc = pl.pallas_call(
    _mid_tc_body,
    out_shape=[
        jax.ShapeDtypeStruct((N, H), _f32),
        jax.ShapeDtypeStruct((N, H), _f32),
    ],
)

_fin_tc = pl.pallas_call(
    _fin_tc_body,
    out_shape=jax.ShapeDtypeStruct((1, 1), _f32),
)


def kernel(x, edge_index, edge_attr, W_enc, b_enc, W_edge, b_edge,
           W_g0, b_g0, W_g1, b_g1, W_g2, b_g2, W_o1, b_o1, W_o2, b_o2):
    del edge_attr, W_edge, b_edge
    src = edge_index[0].astype(jnp.int32)
    dst = edge_index[1].astype(jnp.int32)
    pad = EPAD - src.shape[0]
    src_p = jnp.concatenate(
        [src, jnp.zeros((pad,), jnp.int32)]).reshape(NW, CPW, CHUNK)
    dst_p = jnp.concatenate(
        [dst, jnp.full((pad,), DUMMY, jnp.int32)]).reshape(NW, CPW, CHUNK)

    zeros1 = jnp.zeros((NPAD,), _f32)
    zeros64 = jnp.zeros((RPW, H), _f32)

    cnt = _deg_sc(dst_p, zeros1)
    h, hs, dinv = _enc_tc(x, W_enc, b_enc.reshape(1, H), cnt)

    for W_l, b_l in ((W_g0, b_g0), (W_g1, b_g1), (W_g2, b_g2)):
        agg = _agg_sc(hs, src_p, dst_p, zeros64)
        h, hs = _mid_tc(agg, hs, dinv, W_l, b_l.reshape(1, H))

    return _fin_tc(h, W_o1, b_o1.reshape(1, H // 2), W_o2, b_o2.reshape(1, 1))
